# baseline (speedup 1.0000x reference)
import functools

import jax
import jax.numpy as jnp
from jax import lax
from jax.experimental import pallas as pl
from jax.experimental.pallas import tpu as pltpu
from jax.experimental.pallas import tpu_sc as plsc

N = 10000
D = 128
C = 40

NC = 2
NS = 16
NW = NC * NS
CHUNK = 128
NBUF = 8
PAD_ROWS = 112
NTOT = N + PAD_ROWS
DEG_W = 8

_SLAB = NTOT // NS
_HSTAGE = 624
_TAIL = N - NS * _HSTAGE


def _pad_edges(e):
    block = NW * CHUNK * NBUF
    return ((e + block - 1) // block) * block


def _zero_rows(zbuf, n_rows):
    zv = jnp.zeros((16,), jnp.float32)

    def body(r, carry):
        for c in (0, 16, C - 16):
            zbuf[r, pl.ds(c, 16)] = zv
        return carry

    lax.fori_loop(0, n_rows, body, 0)


def _add_rows(a, b, dst, cnt):
    def body(r, carry):
        for c in (0, 16, C - 16):
            dst[r, pl.ds(c, 16)] = a[r, pl.ds(c, 16)] + b[r, pl.ds(c, 16)]
        return carry

    lax.fori_loop(0, cnt, body, 0)


def _scale_rows(src, dst, sv, scol, srow0, cnt):
    def body(r, carry):
        srow = sv[srow0 + r, pl.ds(0, 16)]
        d = jnp.broadcast_to(srow[scol], (16,))
        for c in (0, 16, C - 16):
            dst[r, pl.ds(c, 16)] = src[r, pl.ds(c, 16)] * d
        return carry

    lax.fori_loop(0, cnt, body, 0)


@functools.lru_cache(maxsize=None)
def _make_sc_degree(e_pad):
    per_w = e_pad // NW
    n_chunks = per_w // CHUNK
    lag = 8
    mesh = plsc.VectorSubcoreMesh(core_axis_name="c", subcore_axis_name="s")

    @functools.partial(
        pl.kernel,
        mesh=mesh,
        out_type=jax.ShapeDtypeStruct((NC * NTOT, DEG_W), jnp.float32),
        compiler_params=pltpu.CompilerParams(use_tc_tiling_on_sc=False),
        scratch_types=[
            pltpu.VMEM((n_chunks, CHUNK), jnp.int32),
            pltpu.VMEM((CHUNK, DEG_W), jnp.float32),
            pltpu.VMEM((CHUNK, DEG_W), jnp.float32),
            pltpu.VMEM_SHARED((NTOT, DEG_W), jnp.float32),
            pltpu.SemaphoreType.DMA,
        ],
    )
    def deg_kernel(didx_hbm, ones_hbm, zeros_hbm, out_hbm, didx_all, ones_v,
                   zbuf, acc, sem):
        cid = lax.axis_index("c")
        sid = lax.axis_index("s")
        wid = sid * NC + cid
        pltpu.sync_copy(zeros_hbm, zbuf)
        base = sid * _SLAB
        for j in range(4):
            pltpu.sync_copy(zbuf, acc.at[pl.ds(base + j * CHUNK, CHUNK)])
        pltpu.sync_copy(zbuf.at[pl.ds(0, _SLAB - 4 * CHUNK)],
                        acc.at[pl.ds(base + 4 * CHUNK, _SLAB - 4 * CHUNK)])
        pltpu.sync_copy(ones_hbm, ones_v)
        pltpu.sync_copy(didx_hbm.at[wid], didx_all)
        plsc.subcore_barrier()
        for i in range(n_chunks):
            pltpu.async_copy(ones_v, acc.at[didx_all.at[i]], sem, add=True)
            if i >= lag:
                pltpu.make_async_copy(
                    ones_v, acc.at[didx_all.at[i - lag]], sem).wait()
        for i in range(n_chunks - lag, n_chunks):
            pltpu.make_async_copy(ones_v, acc.at[didx_all.at[i]], sem).wait()
        plsc.subcore_barrier()
        pltpu.sync_copy(
            acc.at[pl.ds(sid * _SLAB, _SLAB)],
            out_hbm.at[pl.ds(cid * NTOT + sid * _SLAB, _SLAB)],
        )

    return deg_kernel


@functools.lru_cache(maxsize=None)
def _make_sc_hop(e_pad, first_hop):
    per_w = e_pad // NW
    n_chunks = per_w // CHUNK
    n_groups = n_chunks // NBUF
    scol = 0 if first_hop else 1
    h_shape = (N, C) if first_hop else (NC * NTOT, C)
    mesh = plsc.VectorSubcoreMesh(core_axis_name="c", subcore_axis_name="s")

    @functools.partial(
        pl.kernel,
        mesh=mesh,
        out_type=jax.ShapeDtypeStruct((NC * NTOT, C), jnp.float32),
        compiler_params=pltpu.CompilerParams(use_tc_tiling_on_sc=False),
        scratch_types=[
            pltpu.VMEM((n_chunks, CHUNK), jnp.int32),
            pltpu.VMEM((n_chunks, CHUNK), jnp.int32),
            pltpu.VMEM((NBUF, CHUNK, C), jnp.float32),
            pltpu.VMEM((_HSTAGE + _TAIL, 16), jnp.float32),
            pltpu.VMEM_SHARED((N, C), jnp.float32),
            pltpu.VMEM_SHARED((NTOT, C), jnp.float32),
            pltpu.SemaphoreType.DMA((NBUF,)),
            pltpu.SemaphoreType.DMA((NBUF,)),
        ],
    )
    def hop_kernel(h_hbm, s_hbm, sidx_hbm, didx_hbm, out_hbm,
                   sidx_all, didx_all, rows, sv, h_sp, acc, gsem, ssem):
        cid = lax.axis_index("c")
        sid = lax.axis_index("s")
        wid = sid * NC + cid
        r0 = sid * _HSTAGE

        def row_chunks():
            full, rem = divmod(_HSTAGE, CHUNK)
            ch = [(k * CHUNK, CHUNK) for k in range(full)]
            if rem:
                ch.append((full * CHUNK, rem))
            return ch

        if first_hop:
            pltpu.sync_copy(h_hbm.at[pl.ds(r0, _HSTAGE)],
                            h_sp.at[pl.ds(r0, _HSTAGE)])

            @pl.when(sid == NS - 1)
            def _():
                pltpu.sync_copy(h_hbm.at[pl.ds(N - _TAIL, _TAIL)],
                                h_sp.at[pl.ds(N - _TAIL, _TAIL)])

            @pl.when(cid == 0)
            def _():
                pltpu.sync_copy(h_hbm.at[pl.ds(r0, _HSTAGE)],
                                acc.at[pl.ds(r0, _HSTAGE)])

                @pl.when(sid == NS - 1)
                def _():
                    pltpu.sync_copy(h_hbm.at[pl.ds(N - _TAIL, _TAIL)],
                                    acc.at[pl.ds(N - _TAIL, _TAIL)])
        else:
            v0, v1, vs = rows.at[1], rows.at[2], rows.at[3]

            def sum_chunk(off, cnt):
                rg = r0 + off
                pltpu.sync_copy(h_hbm.at[pl.ds(rg, cnt)],
                                v0.at[pl.ds(0, cnt)])
                pltpu.sync_copy(h_hbm.at[pl.ds(NTOT + rg, cnt)],
                                v1.at[pl.ds(0, cnt)])
                _add_rows(v0, v1, vs, cnt)
                pltpu.sync_copy(vs.at[pl.ds(0, cnt)],
                                h_sp.at[pl.ds(rg, cnt)])

                @pl.when(cid == 0)
                def _():
                    pltpu.sync_copy(vs.at[pl.ds(0, cnt)],
                                    acc.at[pl.ds(rg, cnt)])

            for off, cnt in row_chunks():
                sum_chunk(off, cnt)

            @pl.when(sid == NS - 1)
            def _():
                rg = N - _TAIL
                pltpu.sync_copy(h_hbm.at[pl.ds(rg, _TAIL)],
                                v0.at[pl.ds(0, _TAIL)])
                pltpu.sync_copy(h_hbm.at[pl.ds(NTOT + rg, _TAIL)],
                                v1.at[pl.ds(0, _TAIL)])
                _add_rows(v0, v1, vs, _TAIL)
                pltpu.sync_copy(vs.at[pl.ds(0, _TAIL)],
                                h_sp.at[pl.ds(rg, _TAIL)])

                @pl.when(cid == 0)
                def _():
                    pltpu.sync_copy(vs.at[pl.ds(0, _TAIL)],
                                    acc.at[pl.ds(rg, _TAIL)])

        zb = rows.at[0]
        _zero_rows(zb, CHUNK)

        @pl.when(cid == 1)
        def _():
            for off, cnt in row_chunks():
                pltpu.sync_copy(zb.at[pl.ds(0, cnt)],
                                acc.at[pl.ds(r0 + off, cnt)])

            @pl.when(sid == NS - 1)
            def _():
                pltpu.sync_copy(zb.at[pl.ds(0, _TAIL)],
                                acc.at[pl.ds(N - _TAIL, _TAIL)])

        @pl.when(sid == 0)
        def _():
            pltpu.sync_copy(zb.at[pl.ds(0, PAD_ROWS)],
                            acc.at[pl.ds(N, PAD_ROWS)])

        pltpu.sync_copy(sidx_hbm.at[wid], sidx_all)
        pltpu.sync_copy(didx_hbm.at[wid], didx_all)
        pltpu.sync_copy(s_hbm.at[pl.ds(r0, _HSTAGE)],
                        sv.at[pl.ds(0, _HSTAGE)])

        @pl.when(sid == NS - 1)
        def _():
            pltpu.sync_copy(s_hbm.at[pl.ds(N - _TAIL, _TAIL)],
                            sv.at[pl.ds(_HSTAGE, _TAIL)])

        plsc.subcore_barrier()

        def gather(i, b):
            pltpu.async_copy(h_sp.at[sidx_all.at[i]], rows.at[b], gsem.at[b])

        def gather_wait(i, b):
            pltpu.make_async_copy(
                h_sp.at[sidx_all.at[i]], rows.at[b], gsem.at[b]).wait()

        def scatter(i, b):
            pltpu.async_copy(rows.at[b], acc.at[didx_all.at[i]], ssem.at[b],
                             add=True)

        def scatter_wait(i, b):
            pltpu.make_async_copy(
                rows.at[b], acc.at[didx_all.at[i]], ssem.at[b]).wait()

        for b in range(NBUF):
            gather(b, b)

        def body(g, carry):
            i0 = g * NBUF
            for b in range(NBUF):
                gather_wait(i0 + b, b)
                scatter(i0 + b, b)
            for b in range(NBUF):
                scatter_wait(i0 + b, b)
                gather(i0 + NBUF + b, b)
            return carry

        lax.fori_loop(0, n_groups - 1, body, 0)
        i0 = (n_groups - 1) * NBUF
        for b in range(NBUF):
            gather_wait(i0 + b, b)
            scatter(i0 + b, b)
        for b in range(NBUF):
            scatter_wait(i0 + b, b)
        plsc.subcore_barrier()

        av, bv = rows.at[1], rows.at[2]

        def out_chunk(off, cnt):
            rg = r0 + off
            pltpu.sync_copy(acc.at[pl.ds(rg, cnt)], av.at[pl.ds(0, cnt)])
            _scale_rows(av, bv, sv, scol, off, cnt)
            pltpu.sync_copy(bv.at[pl.ds(0, cnt)],
                            out_hbm.at[pl.ds(cid * NTOT + rg, cnt)])

        for off, cnt in row_chunks():
            out_chunk(off, cnt)

        @pl.when(sid == NS - 1)
        def _():
            rg = N - _TAIL
            pltpu.sync_copy(acc.at[pl.ds(rg, _TAIL)], av.at[pl.ds(0, _TAIL)])
            _scale_rows(av, bv, sv, scol, _HSTAGE, _TAIL)
            pltpu.sync_copy(bv.at[pl.ds(0, _TAIL)],
                            out_hbm.at[pl.ds(cid * NTOT + rg, _TAIL)])

    return hop_kernel


_BR = 2000


def _tc_head(x, W, degp):
    def body(x_ref, w_ref, d0_ref, d1_ref, z_ref, s_ref):
        deg = d0_ref[0, :, 0:1] + d1_ref[0, :, 0:1] + 1.0
        dinv = lax.rsqrt(deg)
        y = jnp.dot(x_ref[...], w_ref[...], preferred_element_type=jnp.float32)
        z_ref[...] = y * dinv
        col = lax.broadcasted_iota(jnp.int32, (_BR, 16), 1)
        s_ref[...] = jnp.where(col == 0, dinv * dinv,
                               jnp.broadcast_to(dinv, (_BR, 16)))

    return pl.pallas_call(
        body,
        grid=(N // _BR,),
        in_specs=[
            pl.BlockSpec((_BR, D), lambda i: (i, 0)),
            pl.BlockSpec((D, C), lambda i: (0, 0)),
            pl.BlockSpec((1, _BR, DEG_W), lambda i: (0, i, 0)),
            pl.BlockSpec((1, _BR, DEG_W), lambda i: (1, i, 0)),
        ],
        out_specs=[
            pl.BlockSpec((_BR, C), lambda i: (i, 0)),
            pl.BlockSpec((_BR, 16), lambda i: (i, 0)),
        ],
        out_shape=[
            jax.ShapeDtypeStruct((N, C), jnp.float32),
            jax.ShapeDtypeStruct((N, 16), jnp.float32),
        ],
    )(x, W, degp, degp)


def _tc_final(p, b2d):
    def body(p0_ref, p1_ref, b_ref, o_ref):
        t = p0_ref[0] + p1_ref[0] + b_ref[0:1, :]
        m = jnp.max(t, axis=1, keepdims=True)
        e = jnp.exp(t - m)
        s = jnp.sum(e, axis=1, keepdims=True)
        o_ref[...] = t - m - jnp.log(s)

    return pl.pallas_call(
        body,
        grid=(N // _BR,),
        in_specs=[
            pl.BlockSpec((1, _BR, C), lambda i: (0, i, 0)),
            pl.BlockSpec((1, _BR, C), lambda i: (1, i, 0)),
            pl.BlockSpec((8, C), lambda i: (0, 0)),
        ],
        out_specs=pl.BlockSpec((_BR, C), lambda i: (i, 0)),
        out_shape=jax.ShapeDtypeStruct((N, C), jnp.float32),
    )(p, p, b2d)


def kernel(x, edge_index, W, b):
    src = edge_index[0]
    dst = edge_index[1]
    e = src.shape[0]
    e_pad = _pad_edges(e)
    pad = e_pad - e
    per_w = e_pad // NW
    n_chunks = per_w // CHUNK
    pad_i = jnp.arange(pad, dtype=jnp.int32)
    src_p = jnp.concatenate([src, (pad_i * 37) % N]).reshape(
        NW, n_chunks, CHUNK)
    dst_p = jnp.concatenate([dst, N + pad_i % PAD_ROWS]).reshape(
        NW, n_chunks, CHUNK)
    ones_rows = jnp.ones((CHUNK, DEG_W), jnp.float32)
    zeros_rows = jnp.zeros((CHUNK, DEG_W), jnp.float32)
    b2d = jnp.broadcast_to(b[None, :], (8, C))

    degp = _make_sc_degree(e_pad)(dst_p, ones_rows, zeros_rows).reshape(
        2, NTOT, DEG_W)
    z0, s12 = _tc_head(x, W, degp)
    p1 = _make_sc_hop(e_pad, True)(z0, s12, src_p, dst_p)
    p2 = _make_sc_hop(e_pad, False)(p1, s12, src_p, dst_p)
    return _tc_final(p2.reshape(2, NTOT, C), b2d)


# speedup vs baseline: 1.0395x; 1.0395x over previous
import functools

import jax
import jax.numpy as jnp
from jax import lax
from jax.experimental import pallas as pl
from jax.experimental.pallas import tpu as pltpu
from jax.experimental.pallas import tpu_sc as plsc

N = 10000
D = 128
C = 40

NC = 2
NS = 16
NW = NC * NS
CHUNK = 128
NBUF = 8
PAD_ROWS = 112
NTOT = N + PAD_ROWS
DEG_W = 8

_SLAB = NTOT // NS
_HSTAGE = 624
_TAIL = N - NS * _HSTAGE


def _pad_edges(e):
    block = NW * CHUNK * NBUF
    return ((e + block - 1) // block) * block


def _zero_rows(zbuf, n_rows):
    zv = jnp.zeros((16,), jnp.float32)

    def body(r, carry):
        for c in (0, 16, C - 16):
            zbuf[r, pl.ds(c, 16)] = zv
        return carry

    lax.fori_loop(0, n_rows, body, 0)


def _add_rows(a, b, dst, cnt):
    def body(r, carry):
        for c in (0, 16, C - 16):
            dst[r, pl.ds(c, 16)] = a[r, pl.ds(c, 16)] + b[r, pl.ds(c, 16)]
        return carry

    lax.fori_loop(0, cnt, body, 0)


def _scale_rows(src, dst, sv, scol, srow0, cnt):
    def body(r, carry):
        srow = sv[srow0 + r, pl.ds(0, 16)]
        d = jnp.broadcast_to(srow[scol], (16,))
        for c in (0, 16, C - 16):
            dst[r, pl.ds(c, 16)] = src[r, pl.ds(c, 16)] * d
        return carry

    lax.fori_loop(0, cnt, body, 0)


@functools.lru_cache(maxsize=None)
def _make_sc_degree(e_pad):
    per_w = e_pad // NW
    n_chunks = per_w // CHUNK
    lag = 8
    mesh = plsc.VectorSubcoreMesh(core_axis_name="c", subcore_axis_name="s")

    @functools.partial(
        pl.kernel,
        mesh=mesh,
        out_type=jax.ShapeDtypeStruct((NC * NTOT, DEG_W), jnp.float32),
        compiler_params=pltpu.CompilerParams(use_tc_tiling_on_sc=False),
        scratch_types=[
            pltpu.VMEM((n_chunks, CHUNK), jnp.int32),
            pltpu.VMEM((CHUNK, DEG_W), jnp.float32),
            pltpu.VMEM((CHUNK, DEG_W), jnp.float32),
            pltpu.VMEM_SHARED((NTOT, DEG_W), jnp.float32),
            pltpu.SemaphoreType.DMA,
        ],
    )
    def deg_kernel(didx_hbm, ones_hbm, zeros_hbm, out_hbm, didx_all, ones_v,
                   zbuf, acc, sem):
        cid = lax.axis_index("c")
        sid = lax.axis_index("s")
        wid = sid * NC + cid
        pltpu.sync_copy(zeros_hbm, zbuf)
        base = sid * _SLAB
        for j in range(4):
            pltpu.sync_copy(zbuf, acc.at[pl.ds(base + j * CHUNK, CHUNK)])
        pltpu.sync_copy(zbuf.at[pl.ds(0, _SLAB - 4 * CHUNK)],
                        acc.at[pl.ds(base + 4 * CHUNK, _SLAB - 4 * CHUNK)])
        pltpu.sync_copy(ones_hbm, ones_v)
        pltpu.sync_copy(didx_hbm.at[wid], didx_all)
        plsc.subcore_barrier()
        for i in range(n_chunks):
            pltpu.async_copy(ones_v, acc.at[didx_all.at[i]], sem, add=True)
            if i >= lag:
                pltpu.make_async_copy(
                    ones_v, acc.at[didx_all.at[i - lag]], sem).wait()
        for i in range(n_chunks - lag, n_chunks):
            pltpu.make_async_copy(ones_v, acc.at[didx_all.at[i]], sem).wait()
        plsc.subcore_barrier()
        pltpu.sync_copy(
            acc.at[pl.ds(sid * _SLAB, _SLAB)],
            out_hbm.at[pl.ds(cid * NTOT + sid * _SLAB, _SLAB)],
        )

    return deg_kernel


@functools.lru_cache(maxsize=None)
def _make_sc_hop(e_pad, first_hop):
    per_w = e_pad // NW
    n_chunks = per_w // CHUNK
    n_groups = n_chunks // NBUF
    scol = 0 if first_hop else 1
    h_shape = (N, C) if first_hop else (NC * NTOT, C)
    mesh = plsc.VectorSubcoreMesh(core_axis_name="c", subcore_axis_name="s")

    @functools.partial(
        pl.kernel,
        mesh=mesh,
        out_type=jax.ShapeDtypeStruct((NC * NTOT, C), jnp.float32),
        compiler_params=pltpu.CompilerParams(use_tc_tiling_on_sc=False),
        scratch_types=[
            pltpu.VMEM((n_chunks, CHUNK), jnp.int32),
            pltpu.VMEM((n_chunks, CHUNK), jnp.int32),
            pltpu.VMEM((NBUF, CHUNK, C), jnp.float32),
            pltpu.VMEM((_HSTAGE + _TAIL, 16), jnp.float32),
            pltpu.VMEM_SHARED((N, C), jnp.float32),
            pltpu.VMEM_SHARED((NTOT, C), jnp.float32),
            pltpu.SemaphoreType.DMA((NBUF,)),
            pltpu.SemaphoreType.DMA((NBUF,)),
        ],
    )
    def hop_kernel(h_hbm, s_hbm, sidx_hbm, didx_hbm, out_hbm,
                   sidx_all, didx_all, rows, sv, h_sp, acc, gsem, ssem):
        cid = lax.axis_index("c")
        sid = lax.axis_index("s")
        wid = sid * NC + cid
        r0 = sid * _HSTAGE

        def row_chunks():
            full, rem = divmod(_HSTAGE, CHUNK)
            ch = [(k * CHUNK, CHUNK) for k in range(full)]
            if rem:
                ch.append((full * CHUNK, rem))
            return ch

        if first_hop:
            pltpu.sync_copy(h_hbm.at[pl.ds(r0, _HSTAGE)],
                            h_sp.at[pl.ds(r0, _HSTAGE)])

            @pl.when(sid == NS - 1)
            def _():
                pltpu.sync_copy(h_hbm.at[pl.ds(N - _TAIL, _TAIL)],
                                h_sp.at[pl.ds(N - _TAIL, _TAIL)])

            @pl.when(cid == 0)
            def _():
                pltpu.sync_copy(h_hbm.at[pl.ds(r0, _HSTAGE)],
                                acc.at[pl.ds(r0, _HSTAGE)])

                @pl.when(sid == NS - 1)
                def _():
                    pltpu.sync_copy(h_hbm.at[pl.ds(N - _TAIL, _TAIL)],
                                    acc.at[pl.ds(N - _TAIL, _TAIL)])
        else:
            v0, v1, vs = rows.at[1], rows.at[2], rows.at[3]

            def sum_chunk(off, cnt):
                rg = r0 + off
                pltpu.sync_copy(h_hbm.at[pl.ds(rg, cnt)],
                                v0.at[pl.ds(0, cnt)])
                pltpu.sync_copy(h_hbm.at[pl.ds(NTOT + rg, cnt)],
                                v1.at[pl.ds(0, cnt)])
                _add_rows(v0, v1, vs, cnt)
                pltpu.sync_copy(vs.at[pl.ds(0, cnt)],
                                h_sp.at[pl.ds(rg, cnt)])

                @pl.when(cid == 0)
                def _():
                    pltpu.sync_copy(vs.at[pl.ds(0, cnt)],
                                    acc.at[pl.ds(rg, cnt)])

            for off, cnt in row_chunks():
                sum_chunk(off, cnt)

            @pl.when(sid == NS - 1)
            def _():
                rg = N - _TAIL
                pltpu.sync_copy(h_hbm.at[pl.ds(rg, _TAIL)],
                                v0.at[pl.ds(0, _TAIL)])
                pltpu.sync_copy(h_hbm.at[pl.ds(NTOT + rg, _TAIL)],
                                v1.at[pl.ds(0, _TAIL)])
                _add_rows(v0, v1, vs, _TAIL)
                pltpu.sync_copy(vs.at[pl.ds(0, _TAIL)],
                                h_sp.at[pl.ds(rg, _TAIL)])

                @pl.when(cid == 0)
                def _():
                    pltpu.sync_copy(vs.at[pl.ds(0, _TAIL)],
                                    acc.at[pl.ds(rg, _TAIL)])

        zb = rows.at[0]
        _zero_rows(zb, CHUNK)

        @pl.when(cid == 1)
        def _():
            for off, cnt in row_chunks():
                pltpu.sync_copy(zb.at[pl.ds(0, cnt)],
                                acc.at[pl.ds(r0 + off, cnt)])

            @pl.when(sid == NS - 1)
            def _():
                pltpu.sync_copy(zb.at[pl.ds(0, _TAIL)],
                                acc.at[pl.ds(N - _TAIL, _TAIL)])

        @pl.when(sid == 0)
        def _():
            pltpu.sync_copy(zb.at[pl.ds(0, PAD_ROWS)],
                            acc.at[pl.ds(N, PAD_ROWS)])

        pltpu.sync_copy(sidx_hbm.at[wid], sidx_all)
        pltpu.sync_copy(didx_hbm.at[wid], didx_all)
        pltpu.sync_copy(s_hbm.at[pl.ds(r0, _HSTAGE)],
                        sv.at[pl.ds(0, _HSTAGE)])

        @pl.when(sid == NS - 1)
        def _():
            pltpu.sync_copy(s_hbm.at[pl.ds(N - _TAIL, _TAIL)],
                            sv.at[pl.ds(_HSTAGE, _TAIL)])

        plsc.subcore_barrier()

        def gather(i, b):
            pltpu.async_copy(h_sp.at[sidx_all.at[i]], rows.at[b], gsem.at[b])

        def gather_wait(i, b):
            pltpu.make_async_copy(
                h_sp.at[sidx_all.at[i]], rows.at[b], gsem.at[b]).wait()

        def scatter(i, b):
            pltpu.async_copy(rows.at[b], acc.at[didx_all.at[i]], ssem.at[b],
                             add=True)

        def scatter_wait(i, b):
            pltpu.make_async_copy(
                rows.at[b], acc.at[didx_all.at[i]], ssem.at[b]).wait()

        for b in range(NBUF):
            gather(b, b)

        def body(g, carry):
            i0 = g * NBUF
            for b in range(NBUF):
                gather_wait(i0 + b, b)
                scatter(i0 + b, b)
            for b in range(NBUF):
                scatter_wait(i0 + b, b)
                gather(i0 + NBUF + b, b)
            return carry

        lax.fori_loop(0, n_groups - 1, body, 0)
        i0 = (n_groups - 1) * NBUF
        for b in range(NBUF):
            gather_wait(i0 + b, b)
            scatter(i0 + b, b)
        for b in range(NBUF):
            scatter_wait(i0 + b, b)
        plsc.subcore_barrier()

        av, bv = rows.at[1], rows.at[2]

        def out_chunk(off, cnt):
            rg = r0 + off
            pltpu.sync_copy(acc.at[pl.ds(rg, cnt)], av.at[pl.ds(0, cnt)])
            _scale_rows(av, bv, sv, scol, off, cnt)
            pltpu.sync_copy(bv.at[pl.ds(0, cnt)],
                            out_hbm.at[pl.ds(cid * NTOT + rg, cnt)])

        for off, cnt in row_chunks():
            out_chunk(off, cnt)

        @pl.when(sid == NS - 1)
        def _():
            rg = N - _TAIL
            pltpu.sync_copy(acc.at[pl.ds(rg, _TAIL)], av.at[pl.ds(0, _TAIL)])
            _scale_rows(av, bv, sv, scol, _HSTAGE, _TAIL)
            pltpu.sync_copy(bv.at[pl.ds(0, _TAIL)],
                            out_hbm.at[pl.ds(cid * NTOT + rg, _TAIL)])

    return hop_kernel


_BR = 2000


def _tc_head(x, W, degp):
    def body(x_ref, w_ref, d0_ref, d1_ref, z_ref, s_ref):
        deg = d0_ref[0, :, 0:1] + d1_ref[0, :, 0:1] + 1.0
        dinv = lax.rsqrt(deg)
        y = jnp.dot(x_ref[...], w_ref[...], preferred_element_type=jnp.float32)
        z_ref[...] = y * dinv
        col = lax.broadcasted_iota(jnp.int32, (_BR, 16), 1)
        s_ref[...] = jnp.where(col == 0, dinv * dinv,
                               jnp.broadcast_to(dinv, (_BR, 16)))

    return pl.pallas_call(
        body,
        grid=(N // _BR,),
        in_specs=[
            pl.BlockSpec((_BR, D), lambda i: (i, 0)),
            pl.BlockSpec((D, C), lambda i: (0, 0)),
            pl.BlockSpec((1, _BR, DEG_W), lambda i: (0, i, 0)),
            pl.BlockSpec((1, _BR, DEG_W), lambda i: (1, i, 0)),
        ],
        out_specs=[
            pl.BlockSpec((_BR, C), lambda i: (i, 0)),
            pl.BlockSpec((_BR, 16), lambda i: (i, 0)),
        ],
        out_shape=[
            jax.ShapeDtypeStruct((N, C), jnp.float32),
            jax.ShapeDtypeStruct((N, 16), jnp.float32),
        ],
    )(x, W, degp, degp)


def _tc_final(p, b2d):
    def body(p0_ref, p1_ref, b_ref, o_ref):
        t = p0_ref[0] + p1_ref[0] + b_ref[0:1, :]
        m = jnp.max(t, axis=1, keepdims=True)
        e = jnp.exp(t - m)
        s = jnp.sum(e, axis=1, keepdims=True)
        o_ref[...] = t - m - jnp.log(s)

    return pl.pallas_call(
        body,
        grid=(N // _BR,),
        in_specs=[
            pl.BlockSpec((1, _BR, C), lambda i: (0, i, 0)),
            pl.BlockSpec((1, _BR, C), lambda i: (1, i, 0)),
            pl.BlockSpec((8, C), lambda i: (0, 0)),
        ],
        out_specs=pl.BlockSpec((_BR, C), lambda i: (i, 0)),
        out_shape=jax.ShapeDtypeStruct((N, C), jnp.float32),
    )(p, p, b2d)


def _tc_repack(edge_index, e_pad):
    e = edge_index.shape[1]
    bk = 32768
    grid = e_pad // bk
    rows_b = bk // CHUNK

    def body(e_ref, s_ref, d_ref):
        i = pl.program_id(0)
        r2 = lax.broadcasted_iota(jnp.int32, (rows_b, CHUNK), 0)
        l2 = lax.broadcasted_iota(jnp.int32, (rows_b, CHUNK), 1)
        flat = i * bk + r2 * CHUNK + l2
        real = flat < e
        p = flat - e
        s_ref[...] = jnp.where(real, e_ref[0, :].reshape(rows_b, CHUNK),
                               (p * 37) % N)
        d_ref[...] = jnp.where(real, e_ref[1, :].reshape(rows_b, CHUNK),
                               N + p % PAD_ROWS)

    return pl.pallas_call(
        body,
        grid=(grid,),
        in_specs=[pl.BlockSpec((2, bk), lambda i: (0, i))],
        out_specs=[
            pl.BlockSpec((rows_b, CHUNK), lambda i: (i, 0)),
            pl.BlockSpec((rows_b, CHUNK), lambda i: (i, 0)),
        ],
        out_shape=[
            jax.ShapeDtypeStruct((e_pad // CHUNK, CHUNK), jnp.int32),
            jax.ShapeDtypeStruct((e_pad // CHUNK, CHUNK), jnp.int32),
        ],
    )(edge_index)


def kernel(x, edge_index, W, b):
    e = edge_index.shape[1]
    e_pad = _pad_edges(e)
    per_w = e_pad // NW
    n_chunks = per_w // CHUNK
    src_r, dst_r = _tc_repack(edge_index, e_pad)
    src_p = src_r.reshape(NW, n_chunks, CHUNK)
    dst_p = dst_r.reshape(NW, n_chunks, CHUNK)
    ones_rows = jnp.ones((CHUNK, DEG_W), jnp.float32)
    zeros_rows = jnp.zeros((CHUNK, DEG_W), jnp.float32)
    b2d = jnp.broadcast_to(b[None, :], (8, C))

    degp = _make_sc_degree(e_pad)(dst_p, ones_rows, zeros_rows).reshape(
        2, NTOT, DEG_W)
    z0, s12 = _tc_head(x, W, degp)
    p1 = _make_sc_hop(e_pad, True)(z0, s12, src_p, dst_p)
    p2 = _make_sc_hop(e_pad, False)(p1, s12, src_p, dst_p)
    return _tc_final(p2.reshape(2, NTOT, C), b2d)


# speedup vs baseline: 1.0919x; 1.0504x over previous
import functools

import jax
import jax.numpy as jnp
from jax import lax
from jax.experimental import pallas as pl
from jax.experimental.pallas import tpu as pltpu
from jax.experimental.pallas import tpu_sc as plsc

N = 10000
D = 128
C = 40

NC = 2
NS = 16
NW = NC * NS
CHUNK = 128
NBUF = 8
PAD_ROWS = 112
NTOT = N + PAD_ROWS
DEG_W = 8

_SLAB = NTOT // NS
_HSTAGE = 624
_TAIL = N - NS * _HSTAGE


def _pad_edges(e):
    block = NW * CHUNK * NBUF
    return ((e + block - 1) // block) * block


def _zero_rows(zbuf, n_rows):
    zv = jnp.zeros((16,), jnp.float32)

    def body(r, carry):
        for c in (0, 16, C - 16):
            zbuf[r, pl.ds(c, 16)] = zv
        return carry

    lax.fori_loop(0, n_rows, body, 0)


def _add_rows(a, b, dst, cnt):
    def body(r, carry):
        for c in (0, 16, C - 16):
            dst[r, pl.ds(c, 16)] = a[r, pl.ds(c, 16)] + b[r, pl.ds(c, 16)]
        return carry

    lax.fori_loop(0, cnt, body, 0)


def _scale_rows(src, dst, sv, scol, srow0, cnt):
    def body(r, carry):
        srow = sv[srow0 + r, pl.ds(0, 16)]
        d = jnp.broadcast_to(srow[scol], (16,))
        for c in (0, 16, C - 16):
            dst[r, pl.ds(c, 16)] = src[r, pl.ds(c, 16)] * d
        return carry

    lax.fori_loop(0, cnt, body, 0)


@functools.lru_cache(maxsize=None)
def _make_sc_degree(e_pad):
    per_w = e_pad // NW
    n_chunks = per_w // CHUNK
    lag = 8
    mesh = plsc.VectorSubcoreMesh(core_axis_name="c", subcore_axis_name="s")

    @functools.partial(
        pl.kernel,
        mesh=mesh,
        out_type=jax.ShapeDtypeStruct((NC * NTOT, DEG_W), jnp.float32),
        compiler_params=pltpu.CompilerParams(use_tc_tiling_on_sc=False),
        scratch_types=[
            pltpu.VMEM((n_chunks, CHUNK), jnp.int32),
            pltpu.VMEM((CHUNK, DEG_W), jnp.float32),
            pltpu.VMEM((CHUNK, DEG_W), jnp.float32),
            pltpu.VMEM_SHARED((NTOT, DEG_W), jnp.float32),
            pltpu.SemaphoreType.DMA,
        ],
    )
    def deg_kernel(didx_hbm, ones_hbm, zeros_hbm, out_hbm, didx_all, ones_v,
                   zbuf, acc, sem):
        cid = lax.axis_index("c")
        sid = lax.axis_index("s")
        wid = sid * NC + cid
        pltpu.sync_copy(zeros_hbm, zbuf)
        base = sid * _SLAB
        for j in range(4):
            pltpu.sync_copy(zbuf, acc.at[pl.ds(base + j * CHUNK, CHUNK)])
        pltpu.sync_copy(zbuf.at[pl.ds(0, _SLAB - 4 * CHUNK)],
                        acc.at[pl.ds(base + 4 * CHUNK, _SLAB - 4 * CHUNK)])
        pltpu.sync_copy(ones_hbm, ones_v)
        pltpu.sync_copy(didx_hbm.at[wid], didx_all)
        plsc.subcore_barrier()
        for i in range(n_chunks):
            pltpu.async_copy(ones_v, acc.at[didx_all.at[i]], sem, add=True)
            if i >= lag:
                pltpu.make_async_copy(
                    ones_v, acc.at[didx_all.at[i - lag]], sem).wait()
        for i in range(n_chunks - lag, n_chunks):
            pltpu.make_async_copy(ones_v, acc.at[didx_all.at[i]], sem).wait()
        plsc.subcore_barrier()
        pltpu.sync_copy(
            acc.at[pl.ds(sid * _SLAB, _SLAB)],
            out_hbm.at[pl.ds(cid * NTOT + sid * _SLAB, _SLAB)],
        )

    return deg_kernel


@functools.lru_cache(maxsize=None)
def _make_sc_hop(e_pad, first_hop):
    per_w = e_pad // NW
    n_chunks = per_w // CHUNK
    n_groups = n_chunks // NBUF
    scol = 0 if first_hop else 1
    h_shape = (N, C) if first_hop else (NC * NTOT, C)
    mesh = plsc.VectorSubcoreMesh(core_axis_name="c", subcore_axis_name="s")

    @functools.partial(
        pl.kernel,
        mesh=mesh,
        out_type=jax.ShapeDtypeStruct((NC * NTOT, C), jnp.float32),
        compiler_params=pltpu.CompilerParams(use_tc_tiling_on_sc=False),
        scratch_types=[
            pltpu.VMEM((n_chunks, CHUNK), jnp.int32),
            pltpu.VMEM((n_chunks, CHUNK), jnp.int32),
            pltpu.VMEM((NBUF, CHUNK, C), jnp.float32),
            pltpu.VMEM((_HSTAGE + _TAIL, 16), jnp.float32),
            pltpu.VMEM_SHARED((N, C), jnp.float32),
            pltpu.VMEM_SHARED((NTOT, C), jnp.float32),
            pltpu.SemaphoreType.DMA((NBUF,)),
            pltpu.SemaphoreType.DMA((NBUF,)),
        ],
    )
    def hop_kernel(h_hbm, s_hbm, sidx_hbm, didx_hbm, out_hbm,
                   sidx_all, didx_all, rows, sv, h_sp, acc, gsem, ssem):
        cid = lax.axis_index("c")
        sid = lax.axis_index("s")
        wid = sid * NC + cid
        r0 = sid * _HSTAGE

        def row_chunks():
            full, rem = divmod(_HSTAGE, CHUNK)
            ch = [(k * CHUNK, CHUNK) for k in range(full)]
            if rem:
                ch.append((full * CHUNK, rem))
            return ch

        if first_hop:
            pltpu.sync_copy(h_hbm.at[pl.ds(r0, _HSTAGE)],
                            h_sp.at[pl.ds(r0, _HSTAGE)])

            @pl.when(sid == NS - 1)
            def _():
                pltpu.sync_copy(h_hbm.at[pl.ds(N - _TAIL, _TAIL)],
                                h_sp.at[pl.ds(N - _TAIL, _TAIL)])

            @pl.when(cid == 0)
            def _():
                pltpu.sync_copy(h_hbm.at[pl.ds(r0, _HSTAGE)],
                                acc.at[pl.ds(r0, _HSTAGE)])

                @pl.when(sid == NS - 1)
                def _():
                    pltpu.sync_copy(h_hbm.at[pl.ds(N - _TAIL, _TAIL)],
                                    acc.at[pl.ds(N - _TAIL, _TAIL)])
        else:
            chunks = row_chunks()

            def rd(k):
                off, cnt = chunks[k]
                a = (k % 2) * 2
                pltpu.async_copy(h_hbm.at[pl.ds(r0 + off, cnt)],
                                 rows.at[a].at[pl.ds(0, cnt)], gsem.at[a])
                pltpu.async_copy(h_hbm.at[pl.ds(NTOT + r0 + off, cnt)],
                                 rows.at[a + 1].at[pl.ds(0, cnt)],
                                 gsem.at[a + 1])

            def rd_wait(k):
                off, cnt = chunks[k]
                a = (k % 2) * 2
                pltpu.make_async_copy(
                    h_hbm.at[pl.ds(r0 + off, cnt)],
                    rows.at[a].at[pl.ds(0, cnt)], gsem.at[a]).wait()
                pltpu.make_async_copy(
                    h_hbm.at[pl.ds(NTOT + r0 + off, cnt)],
                    rows.at[a + 1].at[pl.ds(0, cnt)], gsem.at[a + 1]).wait()

            def wr_wait(k):
                off, cnt = chunks[k]
                ws = 4 + (k % 2)
                pltpu.make_async_copy(
                    rows.at[ws].at[pl.ds(0, cnt)],
                    h_sp.at[pl.ds(r0 + off, cnt)], ssem.at[k % 2]).wait()

                @pl.when(cid == 0)
                def _():
                    pltpu.make_async_copy(
                        rows.at[ws].at[pl.ds(0, cnt)],
                        acc.at[pl.ds(r0 + off, cnt)],
                        ssem.at[2 + k % 2]).wait()

            rd(0)
            for k in range(len(chunks)):
                if k + 1 < len(chunks):
                    rd(k + 1)
                rd_wait(k)
                a = (k % 2) * 2
                ws = 4 + (k % 2)
                if k >= 2:
                    wr_wait(k - 2)
                off, cnt = chunks[k]
                _add_rows(rows.at[a], rows.at[a + 1], rows.at[ws], cnt)
                pltpu.async_copy(rows.at[ws].at[pl.ds(0, cnt)],
                                 h_sp.at[pl.ds(r0 + off, cnt)],
                                 ssem.at[k % 2])

                @pl.when(cid == 0)
                def _():
                    pltpu.async_copy(rows.at[ws].at[pl.ds(0, cnt)],
                                     acc.at[pl.ds(r0 + off, cnt)],
                                     ssem.at[2 + k % 2])
            for k in (len(chunks) - 2, len(chunks) - 1):
                wr_wait(k)
            v0, v1, vs = rows.at[1], rows.at[2], rows.at[3]

            @pl.when(sid == NS - 1)
            def _():
                rg = N - _TAIL
                pltpu.sync_copy(h_hbm.at[pl.ds(rg, _TAIL)],
                                v0.at[pl.ds(0, _TAIL)])
                pltpu.sync_copy(h_hbm.at[pl.ds(NTOT + rg, _TAIL)],
                                v1.at[pl.ds(0, _TAIL)])
                _add_rows(v0, v1, vs, _TAIL)
                pltpu.sync_copy(vs.at[pl.ds(0, _TAIL)],
                                h_sp.at[pl.ds(rg, _TAIL)])

                @pl.when(cid == 0)
                def _():
                    pltpu.sync_copy(vs.at[pl.ds(0, _TAIL)],
                                    acc.at[pl.ds(rg, _TAIL)])

        zb = rows.at[0]
        _zero_rows(zb, CHUNK)

        @pl.when(cid == 1)
        def _():
            for off, cnt in row_chunks():
                pltpu.sync_copy(zb.at[pl.ds(0, cnt)],
                                acc.at[pl.ds(r0 + off, cnt)])

            @pl.when(sid == NS - 1)
            def _():
                pltpu.sync_copy(zb.at[pl.ds(0, _TAIL)],
                                acc.at[pl.ds(N - _TAIL, _TAIL)])

        @pl.when(sid == 0)
        def _():
            pltpu.sync_copy(zb.at[pl.ds(0, PAD_ROWS)],
                            acc.at[pl.ds(N, PAD_ROWS)])

        pltpu.sync_copy(sidx_hbm.at[wid], sidx_all)
        pltpu.sync_copy(didx_hbm.at[wid], didx_all)
        pltpu.sync_copy(s_hbm.at[pl.ds(r0, _HSTAGE)],
                        sv.at[pl.ds(0, _HSTAGE)])

        @pl.when(sid == NS - 1)
        def _():
            pltpu.sync_copy(s_hbm.at[pl.ds(N - _TAIL, _TAIL)],
                            sv.at[pl.ds(_HSTAGE, _TAIL)])

        plsc.subcore_barrier()

        def gather(i, b):
            pltpu.async_copy(h_sp.at[sidx_all.at[i]], rows.at[b], gsem.at[b])

        def gather_wait(i, b):
            pltpu.make_async_copy(
                h_sp.at[sidx_all.at[i]], rows.at[b], gsem.at[b]).wait()

        def scatter(i, b):
            pltpu.async_copy(rows.at[b], acc.at[didx_all.at[i]], ssem.at[b],
                             add=True)

        def scatter_wait(i, b):
            pltpu.make_async_copy(
                rows.at[b], acc.at[didx_all.at[i]], ssem.at[b]).wait()

        for b in range(NBUF):
            gather(b, b)

        def body(g, carry):
            i0 = g * NBUF
            for b in range(NBUF):
                gather_wait(i0 + b, b)
                scatter(i0 + b, b)
            for b in range(NBUF):
                scatter_wait(i0 + b, b)
                gather(i0 + NBUF + b, b)
            return carry

        lax.fori_loop(0, n_groups - 1, body, 0)
        i0 = (n_groups - 1) * NBUF
        for b in range(NBUF):
            gather_wait(i0 + b, b)
            scatter(i0 + b, b)
        for b in range(NBUF):
            scatter_wait(i0 + b, b)
        plsc.subcore_barrier()

        chunks = row_chunks()

        def ord_(k):
            off, cnt = chunks[k]
            pltpu.async_copy(acc.at[pl.ds(r0 + off, cnt)],
                             rows.at[k].at[pl.ds(0, cnt)], gsem.at[k])

        def ord_wait(k):
            off, cnt = chunks[k]
            pltpu.make_async_copy(
                acc.at[pl.ds(r0 + off, cnt)],
                rows.at[k].at[pl.ds(0, cnt)], gsem.at[k]).wait()

        def owr_wait(k):
            off, cnt = chunks[k]
            ob = 5 + (k % 3)
            pltpu.make_async_copy(
                rows.at[ob].at[pl.ds(0, cnt)],
                out_hbm.at[pl.ds(cid * NTOT + r0 + off, cnt)],
                ssem.at[k % 3]).wait()

        for k in range(len(chunks)):
            ord_(k)
        for k in range(len(chunks)):
            ord_wait(k)
            ob = 5 + (k % 3)
            if k >= 3:
                owr_wait(k - 3)
            off, cnt = chunks[k]
            _scale_rows(rows.at[k], rows.at[ob], sv, scol, off, cnt)
            pltpu.async_copy(rows.at[ob].at[pl.ds(0, cnt)],
                             out_hbm.at[pl.ds(cid * NTOT + r0 + off, cnt)],
                             ssem.at[k % 3])
        for k in range(max(0, len(chunks) - 3), len(chunks)):
            owr_wait(k)
        av, bv = rows.at[1], rows.at[2]

        @pl.when(sid == NS - 1)
        def _():
            rg = N - _TAIL
            pltpu.sync_copy(acc.at[pl.ds(rg, _TAIL)], av.at[pl.ds(0, _TAIL)])
            _scale_rows(av, bv, sv, scol, _HSTAGE, _TAIL)
            pltpu.sync_copy(bv.at[pl.ds(0, _TAIL)],
                            out_hbm.at[pl.ds(cid * NTOT + rg, _TAIL)])

    return hop_kernel


_BR = 2000


def _tc_head(x, W, degp):
    def body(x_ref, w_ref, d0_ref, d1_ref, z_ref, s_ref):
        deg = d0_ref[0, :, 0:1] + d1_ref[0, :, 0:1] + 1.0
        dinv = lax.rsqrt(deg)
        y = jnp.dot(x_ref[...], w_ref[...], preferred_element_type=jnp.float32)
        z_ref[...] = y * dinv
        col = lax.broadcasted_iota(jnp.int32, (_BR, 16), 1)
        s_ref[...] = jnp.where(col == 0, dinv * dinv,
                               jnp.broadcast_to(dinv, (_BR, 16)))

    return pl.pallas_call(
        body,
        grid=(N // _BR,),
        in_specs=[
            pl.BlockSpec((_BR, D), lambda i: (i, 0)),
            pl.BlockSpec((D, C), lambda i: (0, 0)),
            pl.BlockSpec((1, _BR, DEG_W), lambda i: (0, i, 0)),
            pl.BlockSpec((1, _BR, DEG_W), lambda i: (1, i, 0)),
        ],
        out_specs=[
            pl.BlockSpec((_BR, C), lambda i: (i, 0)),
            pl.BlockSpec((_BR, 16), lambda i: (i, 0)),
        ],
        out_shape=[
            jax.ShapeDtypeStruct((N, C), jnp.float32),
            jax.ShapeDtypeStruct((N, 16), jnp.float32),
        ],
    )(x, W, degp, degp)


def _tc_final(p, b2d):
    def body(p0_ref, p1_ref, b_ref, o_ref):
        t = p0_ref[0] + p1_ref[0] + b_ref[0:1, :]
        m = jnp.max(t, axis=1, keepdims=True)
        e = jnp.exp(t - m)
        s = jnp.sum(e, axis=1, keepdims=True)
        o_ref[...] = t - m - jnp.log(s)

    return pl.pallas_call(
        body,
        grid=(N // _BR,),
        in_specs=[
            pl.BlockSpec((1, _BR, C), lambda i: (0, i, 0)),
            pl.BlockSpec((1, _BR, C), lambda i: (1, i, 0)),
            pl.BlockSpec((8, C), lambda i: (0, 0)),
        ],
        out_specs=pl.BlockSpec((_BR, C), lambda i: (i, 0)),
        out_shape=jax.ShapeDtypeStruct((N, C), jnp.float32),
    )(p, p, b2d)


def _tc_repack(edge_index, e_pad):
    e = edge_index.shape[1]
    bk = 32768
    grid = e_pad // bk
    rows_b = bk // CHUNK

    def body(e_ref, s_ref, d_ref):
        i = pl.program_id(0)
        r2 = lax.broadcasted_iota(jnp.int32, (rows_b, CHUNK), 0)
        l2 = lax.broadcasted_iota(jnp.int32, (rows_b, CHUNK), 1)
        flat = i * bk + r2 * CHUNK + l2
        real = flat < e
        p = flat - e
        s_ref[...] = jnp.where(real, e_ref[0, :].reshape(rows_b, CHUNK),
                               (p * 37) % N)
        d_ref[...] = jnp.where(real, e_ref[1, :].reshape(rows_b, CHUNK),
                               N + p % PAD_ROWS)

    return pl.pallas_call(
        body,
        grid=(grid,),
        in_specs=[pl.BlockSpec((2, bk), lambda i: (0, i))],
        out_specs=[
            pl.BlockSpec((rows_b, CHUNK), lambda i: (i, 0)),
            pl.BlockSpec((rows_b, CHUNK), lambda i: (i, 0)),
        ],
        out_shape=[
            jax.ShapeDtypeStruct((e_pad // CHUNK, CHUNK), jnp.int32),
            jax.ShapeDtypeStruct((e_pad // CHUNK, CHUNK), jnp.int32),
        ],
    )(edge_index)


def kernel(x, edge_index, W, b):
    e = edge_index.shape[1]
    e_pad = _pad_edges(e)
    per_w = e_pad // NW
    n_chunks = per_w // CHUNK
    src_r, dst_r = _tc_repack(edge_index, e_pad)
    src_p = src_r.reshape(NW, n_chunks, CHUNK)
    dst_p = dst_r.reshape(NW, n_chunks, CHUNK)
    ones_rows = jnp.ones((CHUNK, DEG_W), jnp.float32)
    zeros_rows = jnp.zeros((CHUNK, DEG_W), jnp.float32)
    b2d = jnp.broadcast_to(b[None, :], (8, C))

    degp = _make_sc_degree(e_pad)(dst_p, ones_rows, zeros_rows).reshape(
        2, NTOT, DEG_W)
    z0, s12 = _tc_head(x, W, degp)
    p1 = _make_sc_hop(e_pad, True)(z0, s12, src_p, dst_p)
    p2 = _make_sc_hop(e_pad, False)(p1, s12, src_p, dst_p)
    return _tc_final(p2.reshape(2, NTOT, C), b2d)


# speedup vs baseline: 1.1289x; 1.0339x over previous
import functools

import jax
import jax.numpy as jnp
from jax import lax
from jax.experimental import pallas as pl
from jax.experimental.pallas import tpu as pltpu
from jax.experimental.pallas import tpu_sc as plsc

N = 10000
D = 128
C = 40

NC = 2
NS = 16
NW = NC * NS
CHUNK = 128
NBUF = 8
PAD_ROWS = 112
NTOT = N + PAD_ROWS
DEG_W = 8

_SLAB = NTOT // NS
_HSTAGE = 624
_TAIL = N - NS * _HSTAGE


def _pad_edges(e):
    block = NW * CHUNK * NBUF
    return ((e + block - 1) // block) * block


def _zero_rows(zbuf, n_rows):
    zv = jnp.zeros((16,), jnp.float32)

    def body(r, carry):
        for c in (0, 16, C - 16):
            zbuf[r, pl.ds(c, 16)] = zv
        return carry

    lax.fori_loop(0, n_rows, body, 0)


def _add_rows(a, b, dst, cnt):
    def body(r, carry):
        for c in (0, 16, C - 16):
            dst[r, pl.ds(c, 16)] = a[r, pl.ds(c, 16)] + b[r, pl.ds(c, 16)]
        return carry

    lax.fori_loop(0, cnt, body, 0)


def _scale_rows(src, dst, sv, scol, srow0, cnt):
    def body(r, carry):
        srow = sv[srow0 + r, pl.ds(0, 16)]
        d = jnp.broadcast_to(srow[scol], (16,))
        for c in (0, 16, C - 16):
            dst[r, pl.ds(c, 16)] = src[r, pl.ds(c, 16)] * d
        return carry

    lax.fori_loop(0, cnt, body, 0)


@functools.lru_cache(maxsize=None)
def _make_sc_degree(e_pad):
    per_w = e_pad // NW
    n_chunks = per_w // CHUNK
    lag = 8
    mesh = plsc.VectorSubcoreMesh(core_axis_name="c", subcore_axis_name="s")

    @functools.partial(
        pl.kernel,
        mesh=mesh,
        out_type=jax.ShapeDtypeStruct((NC * NTOT, DEG_W), jnp.float32),
        compiler_params=pltpu.CompilerParams(use_tc_tiling_on_sc=False),
        scratch_types=[
            pltpu.VMEM((n_chunks, CHUNK), jnp.int32),
            pltpu.VMEM((CHUNK, DEG_W), jnp.float32),
            pltpu.VMEM((CHUNK, DEG_W), jnp.float32),
            pltpu.VMEM_SHARED((NTOT, DEG_W), jnp.float32),
            pltpu.SemaphoreType.DMA,
        ],
    )
    def deg_kernel(didx_hbm, ones_hbm, zeros_hbm, out_hbm, didx_all, ones_v,
                   zbuf, acc, sem):
        cid = lax.axis_index("c")
        sid = lax.axis_index("s")
        wid = sid * NC + cid
        pltpu.sync_copy(zeros_hbm, zbuf)
        base = sid * _SLAB
        for j in range(4):
            pltpu.sync_copy(zbuf, acc.at[pl.ds(base + j * CHUNK, CHUNK)])
        pltpu.sync_copy(zbuf.at[pl.ds(0, _SLAB - 4 * CHUNK)],
                        acc.at[pl.ds(base + 4 * CHUNK, _SLAB - 4 * CHUNK)])
        pltpu.sync_copy(ones_hbm, ones_v)
        pltpu.sync_copy(didx_hbm.at[wid], didx_all)
        plsc.subcore_barrier()
        for i in range(n_chunks):
            pltpu.async_copy(ones_v, acc.at[didx_all.at[i]], sem, add=True)
            if i >= lag:
                pltpu.make_async_copy(
                    ones_v, acc.at[didx_all.at[i - lag]], sem).wait()
        for i in range(n_chunks - lag, n_chunks):
            pltpu.make_async_copy(ones_v, acc.at[didx_all.at[i]], sem).wait()
        plsc.subcore_barrier()
        pltpu.sync_copy(
            acc.at[pl.ds(sid * _SLAB, _SLAB)],
            out_hbm.at[pl.ds(cid * NTOT + sid * _SLAB, _SLAB)],
        )

    return deg_kernel


@functools.lru_cache(maxsize=None)
def _make_sc_hop(e_pad, first_hop):
    per_w = e_pad // NW
    n_chunks = per_w // CHUNK
    n_groups = n_chunks // NBUF
    scol = 0 if first_hop else 1
    h_shape = (N, C) if first_hop else (NC * NTOT, C)
    mesh = plsc.VectorSubcoreMesh(core_axis_name="c", subcore_axis_name="s")

    @functools.partial(
        pl.kernel,
        mesh=mesh,
        out_type=jax.ShapeDtypeStruct((NC * NTOT, C), jnp.float32),
        compiler_params=pltpu.CompilerParams(use_tc_tiling_on_sc=False),
        scratch_types=[
            pltpu.VMEM((n_chunks, CHUNK), jnp.int32),
            pltpu.VMEM((n_chunks, CHUNK), jnp.int32),
            pltpu.VMEM((NBUF, CHUNK, C), jnp.float32),
            pltpu.VMEM((_HSTAGE + _TAIL, 16), jnp.float32),
            pltpu.VMEM_SHARED((N, C), jnp.float32),
            pltpu.VMEM_SHARED((NTOT, C), jnp.float32),
            pltpu.SemaphoreType.DMA((NBUF,)),
            pltpu.SemaphoreType.DMA((NBUF,)),
        ],
    )
    def hop_kernel(h_hbm, s_hbm, sidx_hbm, didx_hbm, out_hbm,
                   sidx_all, didx_all, rows, sv, h_sp, acc, gsem, ssem):
        cid = lax.axis_index("c")
        sid = lax.axis_index("s")
        wid = sid * NC + cid
        r0 = sid * _HSTAGE

        def row_chunks():
            full, rem = divmod(_HSTAGE, CHUNK)
            ch = [(k * CHUNK, CHUNK) for k in range(full)]
            if rem:
                ch.append((full * CHUNK, rem))
            return ch

        def stage_pairs():
            prs = [(sidx_hbm.at[wid], sidx_all, gsem.at[4]),
                   (didx_hbm.at[wid], didx_all, gsem.at[5]),
                   (s_hbm.at[pl.ds(r0, _HSTAGE)],
                    sv.at[pl.ds(0, _HSTAGE)], gsem.at[6])]
            return prs

        for a_, b_, m_ in stage_pairs():
            pltpu.async_copy(a_, b_, m_)

        @pl.when(sid == NS - 1)
        def _():
            pltpu.async_copy(s_hbm.at[pl.ds(N - _TAIL, _TAIL)],
                             sv.at[pl.ds(_HSTAGE, _TAIL)], gsem.at[7])

        if first_hop:
            pltpu.async_copy(h_hbm.at[pl.ds(r0, _HSTAGE)],
                             h_sp.at[pl.ds(r0, _HSTAGE)], gsem.at[0])

            @pl.when(sid == NS - 1)
            def _():
                pltpu.async_copy(h_hbm.at[pl.ds(N - _TAIL, _TAIL)],
                                 h_sp.at[pl.ds(N - _TAIL, _TAIL)], gsem.at[1])

            @pl.when(cid == 0)
            def _():
                pltpu.async_copy(h_hbm.at[pl.ds(r0, _HSTAGE)],
                                 acc.at[pl.ds(r0, _HSTAGE)], gsem.at[2])

                @pl.when(sid == NS - 1)
                def _():
                    pltpu.async_copy(h_hbm.at[pl.ds(N - _TAIL, _TAIL)],
                                     acc.at[pl.ds(N - _TAIL, _TAIL)],
                                     gsem.at[3])

            pltpu.make_async_copy(h_hbm.at[pl.ds(r0, _HSTAGE)],
                                  h_sp.at[pl.ds(r0, _HSTAGE)],
                                  gsem.at[0]).wait()

            @pl.when(sid == NS - 1)
            def _():
                pltpu.make_async_copy(
                    h_hbm.at[pl.ds(N - _TAIL, _TAIL)],
                    h_sp.at[pl.ds(N - _TAIL, _TAIL)], gsem.at[1]).wait()

            @pl.when(cid == 0)
            def _():
                pltpu.make_async_copy(h_hbm.at[pl.ds(r0, _HSTAGE)],
                                      acc.at[pl.ds(r0, _HSTAGE)],
                                      gsem.at[2]).wait()

                @pl.when(sid == NS - 1)
                def _():
                    pltpu.make_async_copy(
                        h_hbm.at[pl.ds(N - _TAIL, _TAIL)],
                        acc.at[pl.ds(N - _TAIL, _TAIL)], gsem.at[3]).wait()
        else:
            chunks = row_chunks()

            def rd(k):
                off, cnt = chunks[k]
                a = (k % 2) * 2
                pltpu.async_copy(h_hbm.at[pl.ds(r0 + off, cnt)],
                                 rows.at[a].at[pl.ds(0, cnt)], gsem.at[a])
                pltpu.async_copy(h_hbm.at[pl.ds(NTOT + r0 + off, cnt)],
                                 rows.at[a + 1].at[pl.ds(0, cnt)],
                                 gsem.at[a + 1])

            def rd_wait(k):
                off, cnt = chunks[k]
                a = (k % 2) * 2
                pltpu.make_async_copy(
                    h_hbm.at[pl.ds(r0 + off, cnt)],
                    rows.at[a].at[pl.ds(0, cnt)], gsem.at[a]).wait()
                pltpu.make_async_copy(
                    h_hbm.at[pl.ds(NTOT + r0 + off, cnt)],
                    rows.at[a + 1].at[pl.ds(0, cnt)], gsem.at[a + 1]).wait()

            def wr_wait(k):
                off, cnt = chunks[k]
                ws = 4 + (k % 2)
                pltpu.make_async_copy(
                    rows.at[ws].at[pl.ds(0, cnt)],
                    h_sp.at[pl.ds(r0 + off, cnt)], ssem.at[k % 2]).wait()

                @pl.when(cid == 0)
                def _():
                    pltpu.make_async_copy(
                        rows.at[ws].at[pl.ds(0, cnt)],
                        acc.at[pl.ds(r0 + off, cnt)],
                        ssem.at[2 + k % 2]).wait()

            rd(0)
            for k in range(len(chunks)):
                if k + 1 < len(chunks):
                    rd(k + 1)
                rd_wait(k)
                a = (k % 2) * 2
                ws = 4 + (k % 2)
                if k >= 2:
                    wr_wait(k - 2)
                off, cnt = chunks[k]
                _add_rows(rows.at[a], rows.at[a + 1], rows.at[ws], cnt)
                pltpu.async_copy(rows.at[ws].at[pl.ds(0, cnt)],
                                 h_sp.at[pl.ds(r0 + off, cnt)],
                                 ssem.at[k % 2])

                @pl.when(cid == 0)
                def _():
                    pltpu.async_copy(rows.at[ws].at[pl.ds(0, cnt)],
                                     acc.at[pl.ds(r0 + off, cnt)],
                                     ssem.at[2 + k % 2])
            for k in (len(chunks) - 2, len(chunks) - 1):
                wr_wait(k)
            v0, v1, vs = rows.at[1], rows.at[2], rows.at[3]

            @pl.when(sid == NS - 1)
            def _():
                rg = N - _TAIL
                pltpu.sync_copy(h_hbm.at[pl.ds(rg, _TAIL)],
                                v0.at[pl.ds(0, _TAIL)])
                pltpu.sync_copy(h_hbm.at[pl.ds(NTOT + rg, _TAIL)],
                                v1.at[pl.ds(0, _TAIL)])
                _add_rows(v0, v1, vs, _TAIL)
                pltpu.sync_copy(vs.at[pl.ds(0, _TAIL)],
                                h_sp.at[pl.ds(rg, _TAIL)])

                @pl.when(cid == 0)
                def _():
                    pltpu.sync_copy(vs.at[pl.ds(0, _TAIL)],
                                    acc.at[pl.ds(rg, _TAIL)])

        zb = rows.at[6] if first_hop else rows.at[7]
        _zero_rows(zb, CHUNK)

        @pl.when(cid == 1)
        def _():
            zch = list(enumerate(row_chunks()))
            for k, (off, cnt) in zch:
                pltpu.async_copy(zb.at[pl.ds(0, cnt)],
                                 acc.at[pl.ds(r0 + off, cnt)],
                                 ssem.at[4 + k % 4])
            for k, (off, cnt) in zch:
                pltpu.make_async_copy(zb.at[pl.ds(0, cnt)],
                                      acc.at[pl.ds(r0 + off, cnt)],
                                      ssem.at[4 + k % 4]).wait()

            @pl.when(sid == NS - 1)
            def _():
                pltpu.sync_copy(zb.at[pl.ds(0, _TAIL)],
                                acc.at[pl.ds(N - _TAIL, _TAIL)])

        @pl.when(sid == 0)
        def _():
            pltpu.sync_copy(zb.at[pl.ds(0, PAD_ROWS)],
                            acc.at[pl.ds(N, PAD_ROWS)])

        for a_, b_, m_ in stage_pairs():
            pltpu.make_async_copy(a_, b_, m_).wait()

        @pl.when(sid == NS - 1)
        def _():
            pltpu.make_async_copy(s_hbm.at[pl.ds(N - _TAIL, _TAIL)],
                                  sv.at[pl.ds(_HSTAGE, _TAIL)],
                                  gsem.at[7]).wait()

        plsc.subcore_barrier()

        def gather(i, b):
            pltpu.async_copy(h_sp.at[sidx_all.at[i]], rows.at[b], gsem.at[b])

        def gather_wait(i, b):
            pltpu.make_async_copy(
                h_sp.at[sidx_all.at[i]], rows.at[b], gsem.at[b]).wait()

        def scatter(i, b):
            pltpu.async_copy(rows.at[b], acc.at[didx_all.at[i]], ssem.at[b],
                             add=True)

        def scatter_wait(i, b):
            pltpu.make_async_copy(
                rows.at[b], acc.at[didx_all.at[i]], ssem.at[b]).wait()

        for b in range(NBUF):
            gather(b, b)

        def body(g, carry):
            i0 = g * NBUF
            for b in range(NBUF):
                gather_wait(i0 + b, b)
                scatter(i0 + b, b)
            for b in range(NBUF):
                scatter_wait(i0 + b, b)
                gather(i0 + NBUF + b, b)
            return carry

        lax.fori_loop(0, n_groups - 1, body, 0)
        i0 = (n_groups - 1) * NBUF
        for b in range(NBUF):
            gather_wait(i0 + b, b)
            scatter(i0 + b, b)
        for b in range(NBUF):
            scatter_wait(i0 + b, b)
        plsc.subcore_barrier()

        chunks = row_chunks()

        def ord_(k):
            off, cnt = chunks[k]
            pltpu.async_copy(acc.at[pl.ds(r0 + off, cnt)],
                             rows.at[k].at[pl.ds(0, cnt)], gsem.at[k])

        def ord_wait(k):
            off, cnt = chunks[k]
            pltpu.make_async_copy(
                acc.at[pl.ds(r0 + off, cnt)],
                rows.at[k].at[pl.ds(0, cnt)], gsem.at[k]).wait()

        def owr_wait(k):
            off, cnt = chunks[k]
            ob = 5 + (k % 3)
            pltpu.make_async_copy(
                rows.at[ob].at[pl.ds(0, cnt)],
                out_hbm.at[pl.ds(cid * NTOT + r0 + off, cnt)],
                ssem.at[k % 3]).wait()

        for k in range(len(chunks)):
            ord_(k)
        for k in range(len(chunks)):
            ord_wait(k)
            ob = 5 + (k % 3)
            if k >= 3:
                owr_wait(k - 3)
            off, cnt = chunks[k]
            _scale_rows(rows.at[k], rows.at[ob], sv, scol, off, cnt)
            pltpu.async_copy(rows.at[ob].at[pl.ds(0, cnt)],
                             out_hbm.at[pl.ds(cid * NTOT + r0 + off, cnt)],
                             ssem.at[k % 3])
        for k in range(max(0, len(chunks) - 3), len(chunks)):
            owr_wait(k)
        av, bv = rows.at[1], rows.at[2]

        @pl.when(sid == NS - 1)
        def _():
            rg = N - _TAIL
            pltpu.sync_copy(acc.at[pl.ds(rg, _TAIL)], av.at[pl.ds(0, _TAIL)])
            _scale_rows(av, bv, sv, scol, _HSTAGE, _TAIL)
            pltpu.sync_copy(bv.at[pl.ds(0, _TAIL)],
                            out_hbm.at[pl.ds(cid * NTOT + rg, _TAIL)])

    return hop_kernel


_BR = 2000


def _tc_head(x, W, degp):
    def body(x_ref, w_ref, d0_ref, d1_ref, z_ref, s_ref):
        deg = d0_ref[0, :, 0:1] + d1_ref[0, :, 0:1] + 1.0
        dinv = lax.rsqrt(deg)
        y = jnp.dot(x_ref[...], w_ref[...], preferred_element_type=jnp.float32)
        z_ref[...] = y * dinv
        col = lax.broadcasted_iota(jnp.int32, (_BR, 16), 1)
        s_ref[...] = jnp.where(col == 0, dinv * dinv,
                               jnp.broadcast_to(dinv, (_BR, 16)))

    return pl.pallas_call(
        body,
        grid=(N // _BR,),
        in_specs=[
            pl.BlockSpec((_BR, D), lambda i: (i, 0)),
            pl.BlockSpec((D, C), lambda i: (0, 0)),
            pl.BlockSpec((1, _BR, DEG_W), lambda i: (0, i, 0)),
            pl.BlockSpec((1, _BR, DEG_W), lambda i: (1, i, 0)),
        ],
        out_specs=[
            pl.BlockSpec((_BR, C), lambda i: (i, 0)),
            pl.BlockSpec((_BR, 16), lambda i: (i, 0)),
        ],
        out_shape=[
            jax.ShapeDtypeStruct((N, C), jnp.float32),
            jax.ShapeDtypeStruct((N, 16), jnp.float32),
        ],
    )(x, W, degp, degp)


def _tc_final(p, b2d):
    def body(p0_ref, p1_ref, b_ref, o_ref):
        t = p0_ref[0] + p1_ref[0] + b_ref[0:1, :]
        m = jnp.max(t, axis=1, keepdims=True)
        e = jnp.exp(t - m)
        s = jnp.sum(e, axis=1, keepdims=True)
        o_ref[...] = t - m - jnp.log(s)

    return pl.pallas_call(
        body,
        grid=(N // _BR,),
        in_specs=[
            pl.BlockSpec((1, _BR, C), lambda i: (0, i, 0)),
            pl.BlockSpec((1, _BR, C), lambda i: (1, i, 0)),
            pl.BlockSpec((8, C), lambda i: (0, 0)),
        ],
        out_specs=pl.BlockSpec((_BR, C), lambda i: (i, 0)),
        out_shape=jax.ShapeDtypeStruct((N, C), jnp.float32),
    )(p, p, b2d)


def _tc_repack(edge_index, e_pad):
    e = edge_index.shape[1]
    bk = 32768
    grid = e_pad // bk
    rows_b = bk // CHUNK

    def body(e_ref, s_ref, d_ref):
        i = pl.program_id(0)
        r2 = lax.broadcasted_iota(jnp.int32, (rows_b, CHUNK), 0)
        l2 = lax.broadcasted_iota(jnp.int32, (rows_b, CHUNK), 1)
        flat = i * bk + r2 * CHUNK + l2
        real = flat < e
        p = flat - e
        s_ref[...] = jnp.where(real, e_ref[0, :].reshape(rows_b, CHUNK),
                               (p * 37) % N)
        d_ref[...] = jnp.where(real, e_ref[1, :].reshape(rows_b, CHUNK),
                               N + p % PAD_ROWS)

    return pl.pallas_call(
        body,
        grid=(grid,),
        in_specs=[pl.BlockSpec((2, bk), lambda i: (0, i))],
        out_specs=[
            pl.BlockSpec((rows_b, CHUNK), lambda i: (i, 0)),
            pl.BlockSpec((rows_b, CHUNK), lambda i: (i, 0)),
        ],
        out_shape=[
            jax.ShapeDtypeStruct((e_pad // CHUNK, CHUNK), jnp.int32),
            jax.ShapeDtypeStruct((e_pad // CHUNK, CHUNK), jnp.int32),
        ],
    )(edge_index)


def kernel(x, edge_index, W, b):
    e = edge_index.shape[1]
    e_pad = _pad_edges(e)
    per_w = e_pad // NW
    n_chunks = per_w // CHUNK
    src_r, dst_r = _tc_repack(edge_index, e_pad)
    src_p = src_r.reshape(NW, n_chunks, CHUNK)
    dst_p = dst_r.reshape(NW, n_chunks, CHUNK)
    ones_rows = jnp.ones((CHUNK, DEG_W), jnp.float32)
    zeros_rows = jnp.zeros((CHUNK, DEG_W), jnp.float32)
    b2d = jnp.broadcast_to(b[None, :], (8, C))

    degp = _make_sc_degree(e_pad)(dst_p, ones_rows, zeros_rows).reshape(
        2, NTOT, DEG_W)
    z0, s12 = _tc_head(x, W, degp)
    p1 = _make_sc_hop(e_pad, True)(z0, s12, src_p, dst_p)
    p2 = _make_sc_hop(e_pad, False)(p1, s12, src_p, dst_p)
    return _tc_final(p2.reshape(2, NTOT, C), b2d)


# speedup vs baseline: 1.1366x; 1.0068x over previous
import functools

import jax
import jax.numpy as jnp
from jax import lax
from jax.experimental import pallas as pl
from jax.experimental.pallas import tpu as pltpu
from jax.experimental.pallas import tpu_sc as plsc

N = 10000
D = 128
C = 40

NC = 2
NS = 16
NW = NC * NS
CHUNK = 128
NBUF = 8
PAD_ROWS = 112
NTOT = N + PAD_ROWS
DEG_W = 8

_SLAB = NTOT // NS
_HSTAGE = 624
_TAIL = N - NS * _HSTAGE


def _pad_edges(e):
    block = NW * CHUNK * NBUF
    return ((e + block - 1) // block) * block


def _zero_rows(zbuf, n_rows):
    zv = jnp.zeros((16,), jnp.float32)

    def body(r, carry):
        for c in (0, 16, C - 16):
            zbuf[r, pl.ds(c, 16)] = zv
        return carry

    lax.fori_loop(0, n_rows, body, 0)


def _add_rows(a, b, dst, cnt):
    def body(r, carry):
        for c in (0, 16, C - 16):
            dst[r, pl.ds(c, 16)] = a[r, pl.ds(c, 16)] + b[r, pl.ds(c, 16)]
        return carry

    lax.fori_loop(0, cnt, body, 0)


def _scale_rows(src, dst, sv, scol, srow0, cnt):
    def body(r, carry):
        srow = sv[srow0 + r, pl.ds(0, 16)]
        d = jnp.broadcast_to(srow[scol], (16,))
        for c in (0, 16, C - 16):
            dst[r, pl.ds(c, 16)] = src[r, pl.ds(c, 16)] * d
        return carry

    lax.fori_loop(0, cnt, body, 0)


@functools.lru_cache(maxsize=None)
def _make_sc_degree(e_pad):
    per_w = e_pad // NW
    n_chunks = per_w // CHUNK
    lag = 8
    mesh = plsc.VectorSubcoreMesh(core_axis_name="c", subcore_axis_name="s")

    @functools.partial(
        pl.kernel,
        mesh=mesh,
        out_type=jax.ShapeDtypeStruct((NC * NTOT, DEG_W), jnp.float32),
        compiler_params=pltpu.CompilerParams(use_tc_tiling_on_sc=False),
        scratch_types=[
            pltpu.VMEM((n_chunks, CHUNK), jnp.int32),
            pltpu.VMEM((CHUNK, DEG_W), jnp.float32),
            pltpu.VMEM((CHUNK, DEG_W), jnp.float32),
            pltpu.VMEM_SHARED((NTOT, DEG_W), jnp.float32),
            pltpu.SemaphoreType.DMA((8,)),
        ],
    )
    def deg_kernel(didx_hbm, ones_hbm, zeros_hbm, out_hbm, didx_all, ones_v,
                   zbuf, acc, sems):
        cid = lax.axis_index("c")
        sid = lax.axis_index("s")
        wid = sid * NC + cid
        sem = sems.at[0]
        pltpu.async_copy(zeros_hbm, zbuf, sems.at[1])
        pltpu.async_copy(ones_hbm, ones_v, sems.at[2])
        pltpu.async_copy(didx_hbm.at[wid], didx_all, sems.at[3])
        pltpu.make_async_copy(zeros_hbm, zbuf, sems.at[1]).wait()
        base = sid * _SLAB
        for j in range(4):
            pltpu.async_copy(zbuf, acc.at[pl.ds(base + j * CHUNK, CHUNK)],
                             sems.at[4 + j])
        pltpu.sync_copy(zbuf.at[pl.ds(0, _SLAB - 4 * CHUNK)],
                        acc.at[pl.ds(base + 4 * CHUNK, _SLAB - 4 * CHUNK)])
        for j in range(4):
            pltpu.make_async_copy(zbuf,
                                  acc.at[pl.ds(base + j * CHUNK, CHUNK)],
                                  sems.at[4 + j]).wait()
        pltpu.make_async_copy(ones_hbm, ones_v, sems.at[2]).wait()
        pltpu.make_async_copy(didx_hbm.at[wid], didx_all, sems.at[3]).wait()
        plsc.subcore_barrier()
        for i in range(n_chunks):
            pltpu.async_copy(ones_v, acc.at[didx_all.at[i]], sem, add=True)
            if i >= lag:
                pltpu.make_async_copy(
                    ones_v, acc.at[didx_all.at[i - lag]], sem).wait()
        for i in range(n_chunks - lag, n_chunks):
            pltpu.make_async_copy(ones_v, acc.at[didx_all.at[i]], sem).wait()
        plsc.subcore_barrier()
        pltpu.sync_copy(
            acc.at[pl.ds(sid * _SLAB, _SLAB)],
            out_hbm.at[pl.ds(cid * NTOT + sid * _SLAB, _SLAB)],
        )

    return deg_kernel


@functools.lru_cache(maxsize=None)
def _make_sc_hop(e_pad, first_hop):
    per_w = e_pad // NW
    n_chunks = per_w // CHUNK
    n_groups = n_chunks // NBUF
    scol = 0 if first_hop else 1
    h_shape = (N, C) if first_hop else (NC * NTOT, C)
    mesh = plsc.VectorSubcoreMesh(core_axis_name="c", subcore_axis_name="s")

    @functools.partial(
        pl.kernel,
        mesh=mesh,
        out_type=jax.ShapeDtypeStruct((NC * NTOT, C), jnp.float32),
        compiler_params=pltpu.CompilerParams(use_tc_tiling_on_sc=False),
        scratch_types=[
            pltpu.VMEM((n_chunks, CHUNK), jnp.int32),
            pltpu.VMEM((n_chunks, CHUNK), jnp.int32),
            pltpu.VMEM((NBUF, CHUNK, C), jnp.float32),
            pltpu.VMEM((_HSTAGE + _TAIL, 16), jnp.float32),
            pltpu.VMEM_SHARED((N, C), jnp.float32),
            pltpu.VMEM_SHARED((NTOT, C), jnp.float32),
            pltpu.SemaphoreType.DMA((NBUF,)),
            pltpu.SemaphoreType.DMA((NBUF,)),
        ],
    )
    def hop_kernel(h_hbm, s_hbm, sidx_hbm, didx_hbm, out_hbm,
                   sidx_all, didx_all, rows, sv, h_sp, acc, gsem, ssem):
        cid = lax.axis_index("c")
        sid = lax.axis_index("s")
        wid = sid * NC + cid
        r0 = sid * _HSTAGE

        def row_chunks():
            full, rem = divmod(_HSTAGE, CHUNK)
            ch = [(k * CHUNK, CHUNK) for k in range(full)]
            if rem:
                ch.append((full * CHUNK, rem))
            return ch

        def stage_pairs():
            prs = [(sidx_hbm.at[wid], sidx_all, gsem.at[4]),
                   (didx_hbm.at[wid], didx_all, gsem.at[5]),
                   (s_hbm.at[pl.ds(r0, _HSTAGE)],
                    sv.at[pl.ds(0, _HSTAGE)], gsem.at[6])]
            return prs

        for a_, b_, m_ in stage_pairs():
            pltpu.async_copy(a_, b_, m_)

        @pl.when(sid == NS - 1)
        def _():
            pltpu.async_copy(s_hbm.at[pl.ds(N - _TAIL, _TAIL)],
                             sv.at[pl.ds(_HSTAGE, _TAIL)], gsem.at[7])

        if first_hop:
            pltpu.async_copy(h_hbm.at[pl.ds(r0, _HSTAGE)],
                             h_sp.at[pl.ds(r0, _HSTAGE)], gsem.at[0])

            @pl.when(sid == NS - 1)
            def _():
                pltpu.async_copy(h_hbm.at[pl.ds(N - _TAIL, _TAIL)],
                                 h_sp.at[pl.ds(N - _TAIL, _TAIL)], gsem.at[1])

            @pl.when(cid == 0)
            def _():
                pltpu.async_copy(h_hbm.at[pl.ds(r0, _HSTAGE)],
                                 acc.at[pl.ds(r0, _HSTAGE)], gsem.at[2])

                @pl.when(sid == NS - 1)
                def _():
                    pltpu.async_copy(h_hbm.at[pl.ds(N - _TAIL, _TAIL)],
                                     acc.at[pl.ds(N - _TAIL, _TAIL)],
                                     gsem.at[3])

            pltpu.make_async_copy(h_hbm.at[pl.ds(r0, _HSTAGE)],
                                  h_sp.at[pl.ds(r0, _HSTAGE)],
                                  gsem.at[0]).wait()

            @pl.when(sid == NS - 1)
            def _():
                pltpu.make_async_copy(
                    h_hbm.at[pl.ds(N - _TAIL, _TAIL)],
                    h_sp.at[pl.ds(N - _TAIL, _TAIL)], gsem.at[1]).wait()

            @pl.when(cid == 0)
            def _():
                pltpu.make_async_copy(h_hbm.at[pl.ds(r0, _HSTAGE)],
                                      acc.at[pl.ds(r0, _HSTAGE)],
                                      gsem.at[2]).wait()

                @pl.when(sid == NS - 1)
                def _():
                    pltpu.make_async_copy(
                        h_hbm.at[pl.ds(N - _TAIL, _TAIL)],
                        acc.at[pl.ds(N - _TAIL, _TAIL)], gsem.at[3]).wait()
        else:
            chunks = row_chunks()

            def rd(k):
                off, cnt = chunks[k]
                a = (k % 2) * 2
                pltpu.async_copy(h_hbm.at[pl.ds(r0 + off, cnt)],
                                 rows.at[a].at[pl.ds(0, cnt)], gsem.at[a])
                pltpu.async_copy(h_hbm.at[pl.ds(NTOT + r0 + off, cnt)],
                                 rows.at[a + 1].at[pl.ds(0, cnt)],
                                 gsem.at[a + 1])

            def rd_wait(k):
                off, cnt = chunks[k]
                a = (k % 2) * 2
                pltpu.make_async_copy(
                    h_hbm.at[pl.ds(r0 + off, cnt)],
                    rows.at[a].at[pl.ds(0, cnt)], gsem.at[a]).wait()
                pltpu.make_async_copy(
                    h_hbm.at[pl.ds(NTOT + r0 + off, cnt)],
                    rows.at[a + 1].at[pl.ds(0, cnt)], gsem.at[a + 1]).wait()

            def wr_wait(k):
                off, cnt = chunks[k]
                ws = 4 + (k % 2)
                pltpu.make_async_copy(
                    rows.at[ws].at[pl.ds(0, cnt)],
                    h_sp.at[pl.ds(r0 + off, cnt)], ssem.at[k % 2]).wait()

                @pl.when(cid == 0)
                def _():
                    pltpu.make_async_copy(
                        rows.at[ws].at[pl.ds(0, cnt)],
                        acc.at[pl.ds(r0 + off, cnt)],
                        ssem.at[2 + k % 2]).wait()

            rd(0)
            for k in range(len(chunks)):
                if k + 1 < len(chunks):
                    rd(k + 1)
                rd_wait(k)
                a = (k % 2) * 2
                ws = 4 + (k % 2)
                if k >= 2:
                    wr_wait(k - 2)
                off, cnt = chunks[k]
                _add_rows(rows.at[a], rows.at[a + 1], rows.at[ws], cnt)
                pltpu.async_copy(rows.at[ws].at[pl.ds(0, cnt)],
                                 h_sp.at[pl.ds(r0 + off, cnt)],
                                 ssem.at[k % 2])

                @pl.when(cid == 0)
                def _():
                    pltpu.async_copy(rows.at[ws].at[pl.ds(0, cnt)],
                                     acc.at[pl.ds(r0 + off, cnt)],
                                     ssem.at[2 + k % 2])
            for k in (len(chunks) - 2, len(chunks) - 1):
                wr_wait(k)
            v0, v1, vs = rows.at[1], rows.at[2], rows.at[3]

            @pl.when(sid == NS - 1)
            def _():
                rg = N - _TAIL
                pltpu.sync_copy(h_hbm.at[pl.ds(rg, _TAIL)],
                                v0.at[pl.ds(0, _TAIL)])
                pltpu.sync_copy(h_hbm.at[pl.ds(NTOT + rg, _TAIL)],
                                v1.at[pl.ds(0, _TAIL)])
                _add_rows(v0, v1, vs, _TAIL)
                pltpu.sync_copy(vs.at[pl.ds(0, _TAIL)],
                                h_sp.at[pl.ds(rg, _TAIL)])

                @pl.when(cid == 0)
                def _():
                    pltpu.sync_copy(vs.at[pl.ds(0, _TAIL)],
                                    acc.at[pl.ds(rg, _TAIL)])

        zb = rows.at[6] if first_hop else rows.at[7]
        _zero_rows(zb, CHUNK)

        @pl.when(cid == 1)
        def _():
            zch = list(enumerate(row_chunks()))
            for k, (off, cnt) in zch:
                pltpu.async_copy(zb.at[pl.ds(0, cnt)],
                                 acc.at[pl.ds(r0 + off, cnt)],
                                 ssem.at[4 + k % 4])
            for k, (off, cnt) in zch:
                pltpu.make_async_copy(zb.at[pl.ds(0, cnt)],
                                      acc.at[pl.ds(r0 + off, cnt)],
                                      ssem.at[4 + k % 4]).wait()

            @pl.when(sid == NS - 1)
            def _():
                pltpu.sync_copy(zb.at[pl.ds(0, _TAIL)],
                                acc.at[pl.ds(N - _TAIL, _TAIL)])

        @pl.when(sid == 0)
        def _():
            pltpu.sync_copy(zb.at[pl.ds(0, PAD_ROWS)],
                            acc.at[pl.ds(N, PAD_ROWS)])

        for a_, b_, m_ in stage_pairs():
            pltpu.make_async_copy(a_, b_, m_).wait()

        @pl.when(sid == NS - 1)
        def _():
            pltpu.make_async_copy(s_hbm.at[pl.ds(N - _TAIL, _TAIL)],
                                  sv.at[pl.ds(_HSTAGE, _TAIL)],
                                  gsem.at[7]).wait()

        plsc.subcore_barrier()

        def gather(i, b):
            pltpu.async_copy(h_sp.at[sidx_all.at[i]], rows.at[b], gsem.at[b])

        def gather_wait(i, b):
            pltpu.make_async_copy(
                h_sp.at[sidx_all.at[i]], rows.at[b], gsem.at[b]).wait()

        def scatter(i, b):
            pltpu.async_copy(rows.at[b], acc.at[didx_all.at[i]], ssem.at[b],
                             add=True)

        def scatter_wait(i, b):
            pltpu.make_async_copy(
                rows.at[b], acc.at[didx_all.at[i]], ssem.at[b]).wait()

        for b in range(NBUF):
            gather(b, b)

        def body(g, carry):
            i0 = g * NBUF
            for b in range(NBUF):
                gather_wait(i0 + b, b)
                scatter(i0 + b, b)
            for b in range(NBUF):
                scatter_wait(i0 + b, b)
                gather(i0 + NBUF + b, b)
            return carry

        lax.fori_loop(0, n_groups - 1, body, 0)
        i0 = (n_groups - 1) * NBUF
        for b in range(NBUF):
            gather_wait(i0 + b, b)
            scatter(i0 + b, b)
        for b in range(NBUF):
            scatter_wait(i0 + b, b)
        plsc.subcore_barrier()

        chunks = row_chunks()

        def ord_(k):
            off, cnt = chunks[k]
            pltpu.async_copy(acc.at[pl.ds(r0 + off, cnt)],
                             rows.at[k].at[pl.ds(0, cnt)], gsem.at[k])

        def ord_wait(k):
            off, cnt = chunks[k]
            pltpu.make_async_copy(
                acc.at[pl.ds(r0 + off, cnt)],
                rows.at[k].at[pl.ds(0, cnt)], gsem.at[k]).wait()

        def owr_wait(k):
            off, cnt = chunks[k]
            ob = 5 + (k % 3)
            pltpu.make_async_copy(
                rows.at[ob].at[pl.ds(0, cnt)],
                out_hbm.at[pl.ds(cid * NTOT + r0 + off, cnt)],
                ssem.at[k % 3]).wait()

        for k in range(len(chunks)):
            ord_(k)
        for k in range(len(chunks)):
            ord_wait(k)
            ob = 5 + (k % 3)
            if k >= 3:
                owr_wait(k - 3)
            off, cnt = chunks[k]
            _scale_rows(rows.at[k], rows.at[ob], sv, scol, off, cnt)
            pltpu.async_copy(rows.at[ob].at[pl.ds(0, cnt)],
                             out_hbm.at[pl.ds(cid * NTOT + r0 + off, cnt)],
                             ssem.at[k % 3])
        for k in range(max(0, len(chunks) - 3), len(chunks)):
            owr_wait(k)
        av, bv = rows.at[1], rows.at[2]

        @pl.when(sid == NS - 1)
        def _():
            rg = N - _TAIL
            pltpu.sync_copy(acc.at[pl.ds(rg, _TAIL)], av.at[pl.ds(0, _TAIL)])
            _scale_rows(av, bv, sv, scol, _HSTAGE, _TAIL)
            pltpu.sync_copy(bv.at[pl.ds(0, _TAIL)],
                            out_hbm.at[pl.ds(cid * NTOT + rg, _TAIL)])

    return hop_kernel


_BR = 2000


def _tc_head(x, W, degp):
    def body(x_ref, w_ref, d0_ref, d1_ref, z_ref, s_ref):
        deg = d0_ref[0, :, 0:1] + d1_ref[0, :, 0:1] + 1.0
        dinv = lax.rsqrt(deg)
        y = jnp.dot(x_ref[...], w_ref[...], preferred_element_type=jnp.float32)
        z_ref[...] = y * dinv
        col = lax.broadcasted_iota(jnp.int32, (_BR, 16), 1)
        s_ref[...] = jnp.where(col == 0, dinv * dinv,
                               jnp.broadcast_to(dinv, (_BR, 16)))

    return pl.pallas_call(
        body,
        grid=(N // _BR,),
        in_specs=[
            pl.BlockSpec((_BR, D), lambda i: (i, 0)),
            pl.BlockSpec((D, C), lambda i: (0, 0)),
            pl.BlockSpec((1, _BR, DEG_W), lambda i: (0, i, 0)),
            pl.BlockSpec((1, _BR, DEG_W), lambda i: (1, i, 0)),
        ],
        out_specs=[
            pl.BlockSpec((_BR, C), lambda i: (i, 0)),
            pl.BlockSpec((_BR, 16), lambda i: (i, 0)),
        ],
        out_shape=[
            jax.ShapeDtypeStruct((N, C), jnp.float32),
            jax.ShapeDtypeStruct((N, 16), jnp.float32),
        ],
    )(x, W, degp, degp)


def _tc_final(p, b2d):
    def body(p0_ref, p1_ref, b_ref, o_ref):
        t = p0_ref[0] + p1_ref[0] + b_ref[0:1, :]
        m = jnp.max(t, axis=1, keepdims=True)
        e = jnp.exp(t - m)
        s = jnp.sum(e, axis=1, keepdims=True)
        o_ref[...] = t - m - jnp.log(s)

    return pl.pallas_call(
        body,
        grid=(N // _BR,),
        in_specs=[
            pl.BlockSpec((1, _BR, C), lambda i: (0, i, 0)),
            pl.BlockSpec((1, _BR, C), lambda i: (1, i, 0)),
            pl.BlockSpec((8, C), lambda i: (0, 0)),
        ],
        out_specs=pl.BlockSpec((_BR, C), lambda i: (i, 0)),
        out_shape=jax.ShapeDtypeStruct((N, C), jnp.float32),
    )(p, p, b2d)


def _tc_repack(edge_index, e_pad):
    e = edge_index.shape[1]
    bk = 32768
    grid = e_pad // bk
    rows_b = bk // CHUNK

    def body(e_ref, s_ref, d_ref):
        i = pl.program_id(0)
        r2 = lax.broadcasted_iota(jnp.int32, (rows_b, CHUNK), 0)
        l2 = lax.broadcasted_iota(jnp.int32, (rows_b, CHUNK), 1)
        flat = i * bk + r2 * CHUNK + l2
        real = flat < e
        p = flat - e
        s_ref[...] = jnp.where(real, e_ref[0, :].reshape(rows_b, CHUNK),
                               (p * 37) % N)
        d_ref[...] = jnp.where(real, e_ref[1, :].reshape(rows_b, CHUNK),
                               N + p % PAD_ROWS)

    return pl.pallas_call(
        body,
        grid=(grid,),
        in_specs=[pl.BlockSpec((2, bk), lambda i: (0, i))],
        out_specs=[
            pl.BlockSpec((rows_b, CHUNK), lambda i: (i, 0)),
            pl.BlockSpec((rows_b, CHUNK), lambda i: (i, 0)),
        ],
        out_shape=[
            jax.ShapeDtypeStruct((e_pad // CHUNK, CHUNK), jnp.int32),
            jax.ShapeDtypeStruct((e_pad // CHUNK, CHUNK), jnp.int32),
        ],
    )(edge_index)


def kernel(x, edge_index, W, b):
    e = edge_index.shape[1]
    e_pad = _pad_edges(e)
    per_w = e_pad // NW
    n_chunks = per_w // CHUNK
    src_r, dst_r = _tc_repack(edge_index, e_pad)
    src_p = src_r.reshape(NW, n_chunks, CHUNK)
    dst_p = dst_r.reshape(NW, n_chunks, CHUNK)
    ones_rows = jnp.ones((CHUNK, DEG_W), jnp.float32)
    zeros_rows = jnp.zeros((CHUNK, DEG_W), jnp.float32)
    b2d = jnp.broadcast_to(b[None, :], (8, C))

    degp = _make_sc_degree(e_pad)(dst_p, ones_rows, zeros_rows).reshape(
        2, NTOT, DEG_W)
    z0, s12 = _tc_head(x, W, degp)
    p1 = _make_sc_hop(e_pad, True)(z0, s12, src_p, dst_p)
    p2 = _make_sc_hop(e_pad, False)(p1, s12, src_p, dst_p)
    return _tc_final(p2.reshape(2, NTOT, C), b2d)
